# baseline (speedup 1.0000x reference)
import functools

import jax
import jax.numpy as jnp
from jax.experimental import pallas as pl
from jax.experimental.pallas import tpu as pltpu


def _shift_stack(x):
    z = jnp.zeros((x.shape[0], 1), x.dtype)
    xl = jnp.concatenate([z, x[:, :-1]], axis=1)
    xr = jnp.concatenate([x[:, 1:], z], axis=1)
    return jnp.concatenate([xl, x, xr], axis=0)


def _pass1_kernel(x_ref, w1_ref, h1_ref, sum_ref, sq_ref, *, nb):
    w1c = w1_ref[...]
    for i in range(nb):
        xb = x_ref[i].astype(jnp.bfloat16)
        h1 = jnp.dot(w1c, _shift_stack(xb),
                     preferred_element_type=jnp.float32)
        h1_ref[i] = h1.astype(jnp.bfloat16)
        sum_ref[i] = jnp.sum(h1, axis=1, keepdims=True)
        sq_ref[i] = jnp.sum(h1 * h1, axis=1, keepdims=True)


def _pass2_kernel(h1_ref, w2_ref, s1_ref, t1_ref, h2_ref, sum_ref, sq_ref, *, nb):
    w2c = w2_ref[...]
    s1 = s1_ref[...]
    t1 = t1_ref[...]
    for i in range(nb):
        h1 = h1_ref[i].astype(jnp.float32)
        a1 = jnp.maximum(h1 * s1 + t1, 0.0).astype(jnp.bfloat16)
        h2 = jnp.dot(w2c, _shift_stack(a1),
                     preferred_element_type=jnp.float32)
        h2_ref[i] = h2.astype(jnp.bfloat16)
        sum_ref[i] = jnp.sum(h2, axis=1, keepdims=True)
        sq_ref[i] = jnp.sum(h2 * h2, axis=1, keepdims=True)


def _pass3_kernel(h2_ref, x_ref, wp_ref, s2_ref, t2_ref, o_ref, *, nb):
    wp = wp_ref[...]
    s2 = s2_ref[...]
    t2 = t2_ref[...]
    for i in range(nb):
        proj = jnp.dot(wp, x_ref[i].astype(jnp.bfloat16),
                       preferred_element_type=jnp.float32)
        z = h2_ref[i].astype(jnp.float32) * s2 + t2
        o_ref[i] = jnp.maximum(z + proj, 0.0)


def _finalize(sums, sqs, gamma, beta, count, eps):
    s = jnp.sum(sums[:, :, 0], axis=0)
    ss = jnp.sum(sqs[:, :, 0], axis=0)
    mean = s / count
    var = jnp.maximum(ss / count - mean * mean, 0.0)
    inv = jax.lax.rsqrt(var + eps)
    scale = gamma.astype(jnp.float32) * inv
    shift = beta.astype(jnp.float32) - mean * scale
    return scale[:, None], shift[:, None]


def kernel(x, w1, g1, b1, w2, g2, b2, wp, eps=1e-5):
    N, Cin, L = x.shape
    Cout = w1.shape[0]
    K = w1.shape[2]
    assert K == 3, "kernel specialized for k=3 'same' convolutions"

    w1c = jnp.transpose(w1, (0, 2, 1)).reshape(Cout, K * Cin).astype(jnp.bfloat16)
    w2c = jnp.transpose(w2, (0, 2, 1)).reshape(Cout, K * Cout).astype(jnp.bfloat16)
    wpc = wp[:, :, 0].astype(jnp.bfloat16)

    nb = next(n for n in (16, 8, 4, 2, 1) if N % n == 0)
    grid = (N // nb,)
    cparams = pltpu.CompilerParams(
        dimension_semantics=("parallel",),
        vmem_limit_bytes=60 * 1024 * 1024,
    )

    def blk(c, l, dtype_shape=None):
        return pl.BlockSpec((nb, c, l), lambda i: (i, 0, 0))

    def rep(shape):
        return pl.BlockSpec(tuple(shape), lambda i: (0,) * len(shape))

    stat_shape = jax.ShapeDtypeStruct((N, Cout, 1), jnp.float32)
    stat_blk = pl.BlockSpec((nb, Cout, 1), lambda i: (i, 0, 0))

    h1, sum1, sq1 = pl.pallas_call(
        functools.partial(_pass1_kernel, nb=nb),
        grid=grid,
        in_specs=[blk(Cin, L), rep(w1c.shape)],
        out_specs=[blk(Cout, L), stat_blk, stat_blk],
        out_shape=[jax.ShapeDtypeStruct((N, Cout, L), jnp.bfloat16),
                   stat_shape, stat_shape],
        compiler_params=cparams,
    )(x, w1c)
    scale1, shift1 = _finalize(sum1, sq1, g1, b1, jnp.float32(N * L), eps)

    h2, sum2, sq2 = pl.pallas_call(
        functools.partial(_pass2_kernel, nb=nb),
        grid=grid,
        in_specs=[blk(Cout, L), rep(w2c.shape), rep((Cout, 1)), rep((Cout, 1))],
        out_specs=[blk(Cout, L), stat_blk, stat_blk],
        out_shape=[jax.ShapeDtypeStruct((N, Cout, L), jnp.bfloat16),
                   stat_shape, stat_shape],
        compiler_params=cparams,
    )(h1, w2c, scale1, shift1)
    scale2, shift2 = _finalize(sum2, sq2, g2, b2, jnp.float32(N * L), eps)

    out = pl.pallas_call(
        functools.partial(_pass3_kernel, nb=nb),
        grid=grid,
        in_specs=[blk(Cout, L), blk(Cin, L), rep(wpc.shape),
                  rep((Cout, 1)), rep((Cout, 1))],
        out_specs=blk(Cout, L),
        out_shape=jax.ShapeDtypeStruct((N, Cout, L), jnp.float32),
        compiler_params=cparams,
    )(h2, x, wpc, scale2, shift2)
    return out


# speedup vs baseline: 1.1666x; 1.1666x over previous
import functools

import jax
import jax.numpy as jnp
from jax.experimental import pallas as pl
from jax.experimental.pallas import tpu as pltpu


def _shift_stack(x):
    z = jnp.zeros((x.shape[0], 1), x.dtype)
    xl = jnp.concatenate([z, x[:, :-1]], axis=1)
    xr = jnp.concatenate([x[:, 1:], z], axis=1)
    return jnp.concatenate([xl, x, xr], axis=0)


def _pass1_kernel(x_ref, w1_ref, wp_ref, h1_ref, pj_ref, sum_ref, sq_ref, *, nb):
    w1c = w1_ref[...]
    wpc = wp_ref[...]
    for i in range(nb):
        xb = x_ref[i].astype(jnp.bfloat16)
        h1 = jnp.dot(w1c, _shift_stack(xb),
                     preferred_element_type=jnp.float32)
        h1_ref[i] = h1.astype(jnp.bfloat16)
        pj_ref[i] = jnp.dot(wpc, xb,
                            preferred_element_type=jnp.float32).astype(jnp.bfloat16)
        sum_ref[i] = jnp.sum(h1, axis=1, keepdims=True)
        sq_ref[i] = jnp.sum(h1 * h1, axis=1, keepdims=True)


def _pass2_kernel(h1_ref, w2_ref, s1_ref, t1_ref, sum_ref, sq_ref, *, nb):
    w2c = w2_ref[...]
    s1 = s1_ref[...]
    t1 = t1_ref[...]
    for i in range(nb):
        h1 = h1_ref[i].astype(jnp.float32)
        a1 = jnp.maximum(h1 * s1 + t1, 0.0).astype(jnp.bfloat16)
        h2 = jnp.dot(w2c, _shift_stack(a1),
                     preferred_element_type=jnp.float32)
        sum_ref[i] = jnp.sum(h2, axis=1, keepdims=True)
        sq_ref[i] = jnp.sum(h2 * h2, axis=1, keepdims=True)


def _pass3_kernel(h1_ref, pj_ref, w2_ref, s1_ref, t1_ref, s2_ref, t2_ref,
                  o_ref, *, nb):
    w2c = w2_ref[...]
    s1 = s1_ref[...]
    t1 = t1_ref[...]
    s2 = s2_ref[...]
    t2 = t2_ref[...]
    for i in range(nb):
        h1 = h1_ref[i].astype(jnp.float32)
        a1 = jnp.maximum(h1 * s1 + t1, 0.0).astype(jnp.bfloat16)
        h2 = jnp.dot(w2c, _shift_stack(a1),
                     preferred_element_type=jnp.float32)
        z = h2 * s2 + t2 + pj_ref[i].astype(jnp.float32)
        o_ref[i] = jnp.maximum(z, 0.0)


def _finalize(sums, sqs, gamma, beta, count, eps):
    s = jnp.sum(sums[:, :, 0], axis=0)
    ss = jnp.sum(sqs[:, :, 0], axis=0)
    mean = s / count
    var = jnp.maximum(ss / count - mean * mean, 0.0)
    inv = jax.lax.rsqrt(var + eps)
    scale = gamma.astype(jnp.float32) * inv
    shift = beta.astype(jnp.float32) - mean * scale
    return scale[:, None], shift[:, None]


def kernel(x, w1, g1, b1, w2, g2, b2, wp, eps=1e-5):
    N, Cin, L = x.shape
    Cout = w1.shape[0]
    K = w1.shape[2]
    assert K == 3, "kernel specialized for k=3 'same' convolutions"

    w1c = jnp.transpose(w1, (0, 2, 1)).reshape(Cout, K * Cin).astype(jnp.bfloat16)
    w2c = jnp.transpose(w2, (0, 2, 1)).reshape(Cout, K * Cout).astype(jnp.bfloat16)
    wpc = wp[:, :, 0].astype(jnp.bfloat16)

    nb = next(n for n in (8, 4, 2, 1) if N % n == 0)
    grid = (N // nb,)
    cparams = pltpu.CompilerParams(
        dimension_semantics=("parallel",),
        vmem_limit_bytes=60 * 1024 * 1024,
    )

    def blk(c, l):
        return pl.BlockSpec((nb, c, l), lambda i: (i, 0, 0))

    def rep(shape):
        return pl.BlockSpec(tuple(shape), lambda i: (0,) * len(shape))

    stat_shape = jax.ShapeDtypeStruct((N, Cout, 1), jnp.float32)
    stat_blk = pl.BlockSpec((nb, Cout, 1), lambda i: (i, 0, 0))
    act_shape = jax.ShapeDtypeStruct((N, Cout, L), jnp.bfloat16)

    h1, pj, sum1, sq1 = pl.pallas_call(
        functools.partial(_pass1_kernel, nb=nb),
        grid=grid,
        in_specs=[blk(Cin, L), rep(w1c.shape), rep(wpc.shape)],
        out_specs=[blk(Cout, L), blk(Cout, L), stat_blk, stat_blk],
        out_shape=[act_shape, act_shape, stat_shape, stat_shape],
        compiler_params=cparams,
    )(x, w1c, wpc)
    scale1, shift1 = _finalize(sum1, sq1, g1, b1, jnp.float32(N * L), eps)

    sum2, sq2 = pl.pallas_call(
        functools.partial(_pass2_kernel, nb=nb),
        grid=grid,
        in_specs=[blk(Cout, L), rep(w2c.shape), rep((Cout, 1)), rep((Cout, 1))],
        out_specs=[stat_blk, stat_blk],
        out_shape=[stat_shape, stat_shape],
        compiler_params=cparams,
    )(h1, w2c, scale1, shift1)
    scale2, shift2 = _finalize(sum2, sq2, g2, b2, jnp.float32(N * L), eps)

    out = pl.pallas_call(
        functools.partial(_pass3_kernel, nb=nb),
        grid=grid,
        in_specs=[blk(Cout, L), blk(Cout, L), rep(w2c.shape),
                  rep((Cout, 1)), rep((Cout, 1)),
                  rep((Cout, 1)), rep((Cout, 1))],
        out_specs=blk(Cout, L),
        out_shape=jax.ShapeDtypeStruct((N, Cout, L), jnp.float32),
        compiler_params=cparams,
    )(h1, pj, w2c, scale1, shift1, scale2, shift2)
    return out


# speedup vs baseline: 1.7028x; 1.4596x over previous
import functools

import jax
import jax.numpy as jnp
from jax.experimental import pallas as pl
from jax.experimental.pallas import tpu as pltpu


def _shift_stack(x):
    z = jnp.zeros((x.shape[0], 1), x.dtype)
    xl = jnp.concatenate([z, x[:, :-1]], axis=1)
    xr = jnp.concatenate([x[:, 1:], z], axis=1)
    return jnp.concatenate([xl, x, xr], axis=0)


def _fused_kernel(x_ref, w1_ref, w2_ref, wp_ref, g1_ref, b1_ref, g2_ref, b2_ref,
                  o_ref, act_ref, pj_ref, sum_ref, sq_ref, sc_ref, sh_ref,
                  *, nb, nc, cnt, eps):
    p = pl.program_id(0)
    c = pl.program_id(1)

    @pl.when((p == 0) & (c == 0))
    def _zero_bn1():
        sum_ref[...] = jnp.zeros_like(sum_ref)
        sq_ref[...] = jnp.zeros_like(sq_ref)

    @pl.when(p == 0)
    def _phase0():
        w1c = w1_ref[...]
        wpc = wp_ref[...]
        acc_s = jnp.zeros_like(sum_ref)
        acc_q = jnp.zeros_like(sq_ref)
        for i in range(nb):
            xb = x_ref[i].astype(jnp.bfloat16)
            h1 = jnp.dot(w1c, _shift_stack(xb),
                         preferred_element_type=jnp.float32)
            act_ref[c * nb + i] = h1.astype(jnp.bfloat16)
            pj_ref[c * nb + i] = jnp.dot(
                wpc, xb, preferred_element_type=jnp.float32).astype(jnp.bfloat16)
            acc_s = acc_s + jnp.sum(h1, axis=1, keepdims=True)
            acc_q = acc_q + jnp.sum(h1 * h1, axis=1, keepdims=True)
        sum_ref[...] += acc_s
        sq_ref[...] += acc_q

    @pl.when((p == 1) & (c == 0))
    def _finalize_bn1():
        mean = sum_ref[...] / cnt
        var = jnp.maximum(sq_ref[...] / cnt - mean * mean, 0.0)
        scale = g1_ref[...] * jax.lax.rsqrt(var + eps)
        sc_ref[...] = scale
        sh_ref[...] = b1_ref[...] - mean * scale
        sum_ref[...] = jnp.zeros_like(sum_ref)
        sq_ref[...] = jnp.zeros_like(sq_ref)

    @pl.when(p == 1)
    def _phase1():
        w2c = w2_ref[...]
        s1 = sc_ref[...]
        t1 = sh_ref[...]
        acc_s = jnp.zeros_like(sum_ref)
        acc_q = jnp.zeros_like(sq_ref)
        for i in range(nb):
            h1 = act_ref[c * nb + i].astype(jnp.float32)
            a1 = jnp.maximum(h1 * s1 + t1, 0.0).astype(jnp.bfloat16)
            h2 = jnp.dot(w2c, _shift_stack(a1),
                         preferred_element_type=jnp.float32)
            act_ref[c * nb + i] = h2.astype(jnp.bfloat16)
            acc_s = acc_s + jnp.sum(h2, axis=1, keepdims=True)
            acc_q = acc_q + jnp.sum(h2 * h2, axis=1, keepdims=True)
        sum_ref[...] += acc_s
        sq_ref[...] += acc_q

    @pl.when((p == 2) & (c == 0))
    def _finalize_bn2():
        mean = sum_ref[...] / cnt
        var = jnp.maximum(sq_ref[...] / cnt - mean * mean, 0.0)
        scale = g2_ref[...] * jax.lax.rsqrt(var + eps)
        sc_ref[...] = scale
        sh_ref[...] = b2_ref[...] - mean * scale

    @pl.when(p == 2)
    def _phase2():
        s2 = sc_ref[...]
        t2 = sh_ref[...]
        for i in range(nb):
            z = act_ref[c * nb + i].astype(jnp.float32) * s2 + t2
            o_ref[i] = jnp.maximum(z + pj_ref[c * nb + i].astype(jnp.float32), 0.0)


def kernel(x, w1, g1, b1, w2, g2, b2, wp, eps=1e-5):
    N, Cin, L = x.shape
    Cout = w1.shape[0]
    K = w1.shape[2]
    assert K == 3, "kernel specialized for k=3 'same' convolutions"

    w1c = jnp.transpose(w1, (0, 2, 1)).reshape(Cout, K * Cin).astype(jnp.bfloat16)
    w2c = jnp.transpose(w2, (0, 2, 1)).reshape(Cout, K * Cout).astype(jnp.bfloat16)
    wpc = wp[:, :, 0].astype(jnp.bfloat16)

    nb = next(n for n in (8, 4, 2, 1) if N % n == 0)
    nc = N // nb
    grid = (3, nc)
    cparams = pltpu.CompilerParams(
        dimension_semantics=("arbitrary", "arbitrary"),
        vmem_limit_bytes=60 * 1024 * 1024,
    )

    def rep3(shape):
        return pl.BlockSpec(tuple(shape), lambda p, c: (0,) * len(shape))

    x_spec = pl.BlockSpec((nb, Cin, L), lambda p, c: (jnp.where(p == 0, c, 0), 0, 0))
    o_spec = pl.BlockSpec((nb, Cout, L), lambda p, c: (jnp.where(p == 2, c, 0), 0, 0))

    out = pl.pallas_call(
        functools.partial(_fused_kernel, nb=nb, nc=nc,
                          cnt=float(N * L), eps=float(eps)),
        grid=grid,
        in_specs=[x_spec, rep3(w1c.shape), rep3(w2c.shape), rep3(wpc.shape),
                  rep3((Cout, 1)), rep3((Cout, 1)), rep3((Cout, 1)), rep3((Cout, 1))],
        out_specs=o_spec,
        out_shape=jax.ShapeDtypeStruct((N, Cout, L), jnp.float32),
        scratch_shapes=[
            pltpu.VMEM((N, Cout, L), jnp.bfloat16),
            pltpu.VMEM((N, Cout, L), jnp.bfloat16),
            pltpu.VMEM((Cout, 1), jnp.float32),
            pltpu.VMEM((Cout, 1), jnp.float32),
            pltpu.VMEM((Cout, 1), jnp.float32),
            pltpu.VMEM((Cout, 1), jnp.float32),
        ],
        compiler_params=cparams,
    )(x, w1c, w2c, wpc,
      g1.astype(jnp.float32)[:, None], b1.astype(jnp.float32)[:, None],
      g2.astype(jnp.float32)[:, None], b2.astype(jnp.float32)[:, None])
    return out


# speedup vs baseline: 1.9021x; 1.1170x over previous
import functools

import jax
import jax.numpy as jnp
from jax.experimental import pallas as pl
from jax.experimental.pallas import tpu as pltpu


def _shift_stack(x):
    z = jnp.zeros((x.shape[0], 1), x.dtype)
    xl = jnp.concatenate([z, x[:, :-1]], axis=1)
    xr = jnp.concatenate([x[:, 1:], z], axis=1)
    return jnp.concatenate([xl, x, xr], axis=0)


def _fold(h, w):
    r = h[:, :w]
    for j in range(1, h.shape[1] // w):
        r = r + h[:, j * w:(j + 1) * w]
    return r


def _fused_kernel(x_ref, w1_ref, w2_ref, wp_ref, g1_ref, b1_ref, g2_ref, b2_ref,
                  o_ref, act_ref, pj_ref, sum_ref, sq_ref, sc_ref, sh_ref,
                  *, nb, nc, cnt, eps, sw):
    p = pl.program_id(0)
    c = pl.program_id(1)

    @pl.when((p == 0) & (c == 0))
    def _zero_bn1():
        sum_ref[...] = jnp.zeros_like(sum_ref)
        sq_ref[...] = jnp.zeros_like(sq_ref)

    @pl.when(p == 0)
    def _phase0():
        w1c = w1_ref[...]
        wpc = wp_ref[...]
        acc_s = jnp.zeros_like(sum_ref)
        acc_q = jnp.zeros_like(sq_ref)
        for i in range(nb):
            xb = x_ref[i].astype(jnp.bfloat16)
            h1 = jnp.dot(w1c, _shift_stack(xb),
                         preferred_element_type=jnp.float32)
            act_ref[c * nb + i] = h1.astype(jnp.bfloat16)
            pj_ref[c * nb + i] = jnp.dot(
                wpc, xb, preferred_element_type=jnp.float32).astype(jnp.bfloat16)
            acc_s = acc_s + _fold(h1, sw)
            acc_q = acc_q + _fold(h1 * h1, sw)
        sum_ref[...] += acc_s
        sq_ref[...] += acc_q

    @pl.when((p == 1) & (c == 0))
    def _finalize_bn1():
        mean = jnp.sum(sum_ref[...], axis=1, keepdims=True) / cnt
        var = jnp.maximum(
            jnp.sum(sq_ref[...], axis=1, keepdims=True) / cnt - mean * mean, 0.0)
        scale = g1_ref[...] * jax.lax.rsqrt(var + eps)
        sc_ref[...] = scale
        sh_ref[...] = b1_ref[...] - mean * scale
        sum_ref[...] = jnp.zeros_like(sum_ref)
        sq_ref[...] = jnp.zeros_like(sq_ref)

    @pl.when(p == 1)
    def _phase1():
        w2c = w2_ref[...]
        s1 = sc_ref[...]
        t1 = sh_ref[...]
        acc_s = jnp.zeros_like(sum_ref)
        acc_q = jnp.zeros_like(sq_ref)
        for i in range(nb):
            h1 = act_ref[c * nb + i].astype(jnp.float32)
            a1 = jnp.maximum(h1 * s1 + t1, 0.0).astype(jnp.bfloat16)
            h2 = jnp.dot(w2c, _shift_stack(a1),
                         preferred_element_type=jnp.float32)
            act_ref[c * nb + i] = h2.astype(jnp.bfloat16)
            acc_s = acc_s + _fold(h2, sw)
            acc_q = acc_q + _fold(h2 * h2, sw)
        sum_ref[...] += acc_s
        sq_ref[...] += acc_q

    @pl.when((p == 2) & (c == 0))
    def _finalize_bn2():
        mean = jnp.sum(sum_ref[...], axis=1, keepdims=True) / cnt
        var = jnp.maximum(
            jnp.sum(sq_ref[...], axis=1, keepdims=True) / cnt - mean * mean, 0.0)
        scale = g2_ref[...] * jax.lax.rsqrt(var + eps)
        sc_ref[...] = scale
        sh_ref[...] = b2_ref[...] - mean * scale

    @pl.when(p == 2)
    def _phase2():
        s2 = sc_ref[...]
        t2 = sh_ref[...]
        for i in range(nb):
            z = act_ref[c * nb + i].astype(jnp.float32) * s2 + t2
            o_ref[i] = jnp.maximum(z + pj_ref[c * nb + i].astype(jnp.float32), 0.0)


def kernel(x, w1, g1, b1, w2, g2, b2, wp, eps=1e-5):
    N, Cin, L = x.shape
    Cout = w1.shape[0]
    K = w1.shape[2]
    assert K == 3, "kernel specialized for k=3 'same' convolutions"

    w1c = jnp.transpose(w1, (0, 2, 1)).reshape(Cout, K * Cin).astype(jnp.bfloat16)
    w2c = jnp.transpose(w2, (0, 2, 1)).reshape(Cout, K * Cout).astype(jnp.bfloat16)
    wpc = wp[:, :, 0].astype(jnp.bfloat16)

    nb = next(n for n in (8, 4, 2, 1) if N % n == 0)
    sw = min(128, L)
    nc = N // nb
    grid = (3, nc)
    cparams = pltpu.CompilerParams(
        dimension_semantics=("arbitrary", "arbitrary"),
        vmem_limit_bytes=60 * 1024 * 1024,
    )

    def rep3(shape):
        return pl.BlockSpec(tuple(shape), lambda p, c: (0,) * len(shape))

    x_spec = pl.BlockSpec((nb, Cin, L), lambda p, c: (jnp.where(p == 0, c, 0), 0, 0))
    o_spec = pl.BlockSpec((nb, Cout, L), lambda p, c: (jnp.where(p == 2, c, 0), 0, 0))

    out = pl.pallas_call(
        functools.partial(_fused_kernel, nb=nb, nc=nc,
                          cnt=float(N * L), eps=float(eps), sw=sw),
        grid=grid,
        in_specs=[x_spec, rep3(w1c.shape), rep3(w2c.shape), rep3(wpc.shape),
                  rep3((Cout, 1)), rep3((Cout, 1)), rep3((Cout, 1)), rep3((Cout, 1))],
        out_specs=o_spec,
        out_shape=jax.ShapeDtypeStruct((N, Cout, L), jnp.float32),
        scratch_shapes=[
            pltpu.VMEM((N, Cout, L), jnp.bfloat16),
            pltpu.VMEM((N, Cout, L), jnp.bfloat16),
            pltpu.VMEM((Cout, sw), jnp.float32),
            pltpu.VMEM((Cout, sw), jnp.float32),
            pltpu.VMEM((Cout, 1), jnp.float32),
            pltpu.VMEM((Cout, 1), jnp.float32),
        ],
        compiler_params=cparams,
    )(x, w1c, w2c, wpc,
      g1.astype(jnp.float32)[:, None], b1.astype(jnp.float32)[:, None],
      g2.astype(jnp.float32)[:, None], b2.astype(jnp.float32)[:, None])
    return out


# speedup vs baseline: 1.9225x; 1.0108x over previous
import functools

import jax
import jax.numpy as jnp
from jax.experimental import pallas as pl
from jax.experimental.pallas import tpu as pltpu


def _shift_stack(x):
    z = jnp.zeros((x.shape[0], 1), x.dtype)
    xl = jnp.concatenate([z, x[:, :-1]], axis=1)
    xr = jnp.concatenate([x[:, 1:], z], axis=1)
    return jnp.concatenate([xl, x, xr], axis=0)


def _fold(h, w):
    r = h[:, :w]
    for j in range(1, h.shape[1] // w):
        r = r + h[:, j * w:(j + 1) * w]
    return r


def _fused_kernel(x_ref, w1_ref, w2_ref, wp_ref, g1_ref, b1_ref, g2_ref, b2_ref,
                  o_ref, act_ref, pj_ref, sum_ref, sq_ref, sc_ref, sh_ref,
                  *, nb, nc, cnt, eps, sw):
    p = pl.program_id(0)
    c = pl.program_id(1)

    @pl.when((p == 0) & (c == 0))
    def _zero_bn1():
        sum_ref[...] = jnp.zeros_like(sum_ref)
        sq_ref[...] = jnp.zeros_like(sq_ref)

    @pl.when(p == 0)
    def _phase0():
        w1c = w1_ref[...]
        wpc = wp_ref[...]
        acc_s = jnp.zeros_like(sum_ref)
        acc_q = jnp.zeros_like(sq_ref)
        for i in range(nb):
            xb = x_ref[i].astype(jnp.bfloat16)
            h1 = jnp.dot(w1c, _shift_stack(xb),
                         preferred_element_type=jnp.float32)
            act_ref[c * nb + i] = h1.astype(jnp.bfloat16)
            pj_ref[c * nb + i] = jnp.dot(
                wpc, xb, preferred_element_type=jnp.float32).astype(jnp.bfloat16)
            acc_s = acc_s + _fold(h1, sw)
            acc_q = acc_q + _fold(h1 * h1, sw)
        sum_ref[...] += acc_s
        sq_ref[...] += acc_q

    @pl.when((p == 1) & (c == 0))
    def _finalize_bn1():
        mean = jnp.sum(sum_ref[...], axis=1, keepdims=True) / cnt
        var = jnp.maximum(
            jnp.sum(sq_ref[...], axis=1, keepdims=True) / cnt - mean * mean, 0.0)
        scale = g1_ref[...] * jax.lax.rsqrt(var + eps)
        sc_ref[...] = scale
        sh_ref[...] = b1_ref[...] - mean * scale
        sum_ref[...] = jnp.zeros_like(sum_ref)
        sq_ref[...] = jnp.zeros_like(sq_ref)

    @pl.when(p == 1)
    def _phase1():
        w2c = w2_ref[...]
        s1 = sc_ref[...].astype(jnp.bfloat16)
        t1 = sh_ref[...].astype(jnp.bfloat16)
        acc_s = jnp.zeros_like(sum_ref)
        acc_q = jnp.zeros_like(sq_ref)
        for i in range(nb):
            h1 = act_ref[c * nb + i]
            a1 = jnp.maximum(h1 * s1 + t1, jnp.bfloat16(0.0))
            h2 = jnp.dot(w2c, _shift_stack(a1),
                         preferred_element_type=jnp.float32)
            act_ref[c * nb + i] = h2.astype(jnp.bfloat16)
            acc_s = acc_s + _fold(h2, sw)
            acc_q = acc_q + _fold(h2 * h2, sw)
        sum_ref[...] += acc_s
        sq_ref[...] += acc_q

    @pl.when((p == 2) & (c == 0))
    def _finalize_bn2():
        mean = jnp.sum(sum_ref[...], axis=1, keepdims=True) / cnt
        var = jnp.maximum(
            jnp.sum(sq_ref[...], axis=1, keepdims=True) / cnt - mean * mean, 0.0)
        scale = g2_ref[...] * jax.lax.rsqrt(var + eps)
        sc_ref[...] = scale
        sh_ref[...] = b2_ref[...] - mean * scale

    @pl.when(p == 2)
    def _phase2():
        s2 = sc_ref[...]
        t2 = sh_ref[...]
        for i in range(nb):
            z = act_ref[c * nb + i].astype(jnp.float32) * s2 + t2
            o_ref[i] = jnp.maximum(z + pj_ref[c * nb + i].astype(jnp.float32), 0.0)


def kernel(x, w1, g1, b1, w2, g2, b2, wp, eps=1e-5):
    N, Cin, L = x.shape
    Cout = w1.shape[0]
    K = w1.shape[2]
    assert K == 3, "kernel specialized for k=3 'same' convolutions"

    w1c = jnp.transpose(w1, (0, 2, 1)).reshape(Cout, K * Cin).astype(jnp.bfloat16)
    w2c = jnp.transpose(w2, (0, 2, 1)).reshape(Cout, K * Cout).astype(jnp.bfloat16)
    wpc = wp[:, :, 0].astype(jnp.bfloat16)

    nb = next(n for n in (8, 4, 2, 1) if N % n == 0)
    sw = min(128, L)
    nc = N // nb
    grid = (3, nc)
    cparams = pltpu.CompilerParams(
        dimension_semantics=("arbitrary", "arbitrary"),
        vmem_limit_bytes=60 * 1024 * 1024,
    )

    def rep3(shape):
        return pl.BlockSpec(tuple(shape), lambda p, c: (0,) * len(shape))

    x_spec = pl.BlockSpec((nb, Cin, L), lambda p, c: (jnp.where(p == 0, c, 0), 0, 0))
    o_spec = pl.BlockSpec((nb, Cout, L), lambda p, c: (jnp.where(p == 2, c, 0), 0, 0))

    out = pl.pallas_call(
        functools.partial(_fused_kernel, nb=nb, nc=nc,
                          cnt=float(N * L), eps=float(eps), sw=sw),
        grid=grid,
        in_specs=[x_spec, rep3(w1c.shape), rep3(w2c.shape), rep3(wpc.shape),
                  rep3((Cout, 1)), rep3((Cout, 1)), rep3((Cout, 1)), rep3((Cout, 1))],
        out_specs=o_spec,
        out_shape=jax.ShapeDtypeStruct((N, Cout, L), jnp.float32),
        scratch_shapes=[
            pltpu.VMEM((N, Cout, L), jnp.bfloat16),
            pltpu.VMEM((N, Cout, L), jnp.bfloat16),
            pltpu.VMEM((Cout, sw), jnp.float32),
            pltpu.VMEM((Cout, sw), jnp.float32),
            pltpu.VMEM((Cout, 1), jnp.float32),
            pltpu.VMEM((Cout, 1), jnp.float32),
        ],
        compiler_params=cparams,
    )(x, w1c, w2c, wpc,
      g1.astype(jnp.float32)[:, None], b1.astype(jnp.float32)[:, None],
      g2.astype(jnp.float32)[:, None], b2.astype(jnp.float32)[:, None])
    return out


# speedup vs baseline: 2.0474x; 1.0650x over previous
import functools

import jax
import jax.numpy as jnp
from jax.experimental import pallas as pl
from jax.experimental.pallas import tpu as pltpu


def _shift_stack(x):
    z = jnp.zeros((x.shape[0], 1), x.dtype)
    xl = jnp.concatenate([z, x[:, :-1]], axis=1)
    xr = jnp.concatenate([x[:, 1:], z], axis=1)
    return jnp.concatenate([xl, x, xr], axis=0)


def _fold(h, w):
    r = h[:, :w]
    for j in range(1, h.shape[1] // w):
        r = r + h[:, j * w:(j + 1) * w]
    return r


def _fused_kernel(x_ref, w1_ref, w2_ref, wp_ref, g1_ref, b1_ref, g2_ref, b2_ref,
                  o_ref, act_ref, xb_ref, sum_ref, sq_ref, sc_ref, sh_ref,
                  *, nb, nc, cnt, eps, sw):
    p = pl.program_id(0)
    c = pl.program_id(1)

    @pl.when((p == 0) & (c == 0))
    def _zero_bn1():
        sum_ref[...] = jnp.zeros_like(sum_ref)
        sq_ref[...] = jnp.zeros_like(sq_ref)

    @pl.when(p == 0)
    def _phase0():
        w1c = w1_ref[...]
        acc_s = jnp.zeros_like(sum_ref)
        acc_q = jnp.zeros_like(sq_ref)
        for i in range(nb):
            xb = x_ref[i].astype(jnp.bfloat16)
            xb_ref[c * nb + i] = xb
            h1 = jnp.dot(w1c, _shift_stack(xb),
                         preferred_element_type=jnp.float32)
            act_ref[c * nb + i] = h1.astype(jnp.bfloat16)
            acc_s = acc_s + _fold(h1, sw)
            acc_q = acc_q + _fold(h1 * h1, sw)
        sum_ref[...] += acc_s
        sq_ref[...] += acc_q

    @pl.when((p == 1) & (c == 0))
    def _finalize_bn1():
        mean = jnp.sum(sum_ref[...], axis=1, keepdims=True) / cnt
        var = jnp.maximum(
            jnp.sum(sq_ref[...], axis=1, keepdims=True) / cnt - mean * mean, 0.0)
        scale = g1_ref[...] * jax.lax.rsqrt(var + eps)
        sc_ref[...] = scale
        sh_ref[...] = b1_ref[...] - mean * scale
        sum_ref[...] = jnp.zeros_like(sum_ref)
        sq_ref[...] = jnp.zeros_like(sq_ref)

    @pl.when(p == 1)
    def _phase1():
        w2c = w2_ref[...]
        s1 = sc_ref[...].astype(jnp.bfloat16)
        t1 = sh_ref[...].astype(jnp.bfloat16)
        acc_s = jnp.zeros_like(sum_ref)
        acc_q = jnp.zeros_like(sq_ref)
        for i in range(nb):
            h1 = act_ref[c * nb + i]
            a1 = jnp.maximum(h1 * s1 + t1, jnp.bfloat16(0.0))
            h2 = jnp.dot(w2c, _shift_stack(a1),
                         preferred_element_type=jnp.float32)
            act_ref[c * nb + i] = h2.astype(jnp.bfloat16)
            acc_s = acc_s + _fold(h2, sw)
            acc_q = acc_q + _fold(h2 * h2, sw)
        sum_ref[...] += acc_s
        sq_ref[...] += acc_q

    @pl.when((p == 2) & (c == 0))
    def _finalize_bn2():
        mean = jnp.sum(sum_ref[...], axis=1, keepdims=True) / cnt
        var = jnp.maximum(
            jnp.sum(sq_ref[...], axis=1, keepdims=True) / cnt - mean * mean, 0.0)
        scale = g2_ref[...] * jax.lax.rsqrt(var + eps)
        sc_ref[...] = scale
        sh_ref[...] = b2_ref[...] - mean * scale

    @pl.when(p == 2)
    def _phase2():
        wpc = wp_ref[...]
        s2 = sc_ref[...]
        t2 = sh_ref[...]
        for i in range(nb):
            proj = jnp.dot(wpc, xb_ref[c * nb + i],
                           preferred_element_type=jnp.float32)
            z = act_ref[c * nb + i].astype(jnp.float32) * s2 + t2
            o_ref[i] = jnp.maximum(z + proj, 0.0)


def kernel(x, w1, g1, b1, w2, g2, b2, wp, eps=1e-5):
    N, Cin, L = x.shape
    Cout = w1.shape[0]
    K = w1.shape[2]
    assert K == 3, "kernel specialized for k=3 'same' convolutions"

    w1c = jnp.transpose(w1, (0, 2, 1)).reshape(Cout, K * Cin).astype(jnp.bfloat16)
    w2c = jnp.transpose(w2, (0, 2, 1)).reshape(Cout, K * Cout).astype(jnp.bfloat16)
    wpc = wp[:, :, 0].astype(jnp.bfloat16)

    nb = next(n for n in (8, 4, 2, 1) if N % n == 0)
    sw = min(128, L)
    nc = N // nb
    grid = (3, nc)
    cparams = pltpu.CompilerParams(
        dimension_semantics=("arbitrary", "arbitrary"),
        vmem_limit_bytes=60 * 1024 * 1024,
    )

    def rep3(shape):
        return pl.BlockSpec(tuple(shape), lambda p, c: (0,) * len(shape))

    x_spec = pl.BlockSpec((nb, Cin, L), lambda p, c: (jnp.where(p == 0, c, 0), 0, 0))
    o_spec = pl.BlockSpec((nb, Cout, L), lambda p, c: (jnp.where(p == 2, c, 0), 0, 0))

    out = pl.pallas_call(
        functools.partial(_fused_kernel, nb=nb, nc=nc,
                          cnt=float(N * L), eps=float(eps), sw=sw),
        grid=grid,
        in_specs=[x_spec, rep3(w1c.shape), rep3(w2c.shape), rep3(wpc.shape),
                  rep3((Cout, 1)), rep3((Cout, 1)), rep3((Cout, 1)), rep3((Cout, 1))],
        out_specs=o_spec,
        out_shape=jax.ShapeDtypeStruct((N, Cout, L), jnp.float32),
        scratch_shapes=[
            pltpu.VMEM((N, Cout, L), jnp.bfloat16),
            pltpu.VMEM((N, Cin, L), jnp.bfloat16),
            pltpu.VMEM((Cout, sw), jnp.float32),
            pltpu.VMEM((Cout, sw), jnp.float32),
            pltpu.VMEM((Cout, 1), jnp.float32),
            pltpu.VMEM((Cout, 1), jnp.float32),
        ],
        compiler_params=cparams,
    )(x, w1c, w2c, wpc,
      g1.astype(jnp.float32)[:, None], b1.astype(jnp.float32)[:, None],
      g2.astype(jnp.float32)[:, None], b2.astype(jnp.float32)[:, None])
    return out


# speedup vs baseline: 2.0709x; 1.0115x over previous
import functools

import jax
import jax.numpy as jnp
from jax.experimental import pallas as pl
from jax.experimental.pallas import tpu as pltpu


def _shift_stack(x):
    z = jnp.zeros((x.shape[0], 1), x.dtype)
    xl = jnp.concatenate([z, x[:, :-1]], axis=1)
    xr = jnp.concatenate([x[:, 1:], z], axis=1)
    return jnp.concatenate([xl, x, xr], axis=0)


def _fold(h, w):
    r = h[:, :w]
    for j in range(1, h.shape[1] // w):
        r = r + h[:, j * w:(j + 1) * w]
    return r


def _fused_kernel(x_ref, w1_ref, w2_ref, wp_ref, g1_ref, b1_ref, g2_ref, b2_ref,
                  o_ref, act_ref, xb_ref, sum_ref, sq_ref, sc_ref, sh_ref,
                  *, nb, nc, cnt, eps, sw):
    s = pl.program_id(0)
    c = s

    @pl.when(s == 0)
    def _zero_bn1():
        sum_ref[...] = jnp.zeros_like(sum_ref)
        sq_ref[...] = jnp.zeros_like(sq_ref)

    @pl.when(s < nc)
    def _phase0():
        w1c = w1_ref[...]
        acc_s = jnp.zeros_like(sum_ref)
        acc_q = jnp.zeros_like(sq_ref)
        for i in range(nb):
            xb = x_ref[i].astype(jnp.bfloat16)
            xb_ref[c * nb + i] = xb
            h1 = jnp.dot(w1c, _shift_stack(xb),
                         preferred_element_type=jnp.float32)
            act_ref[c * nb + i] = h1.astype(jnp.bfloat16)
            acc_s = acc_s + _fold(h1, sw)
            acc_q = acc_q + _fold(h1 * h1, sw)
        sum_ref[...] += acc_s
        sq_ref[...] += acc_q

    @pl.when(s == nc)
    def _phase1():
        mean = jnp.sum(sum_ref[...], axis=1, keepdims=True) / cnt
        var = jnp.maximum(
            jnp.sum(sq_ref[...], axis=1, keepdims=True) / cnt - mean * mean, 0.0)
        scale1 = g1_ref[...] * jax.lax.rsqrt(var + eps)
        s1 = scale1.astype(jnp.bfloat16)
        t1 = (b1_ref[...] - mean * scale1).astype(jnp.bfloat16)

        w2c = w2_ref[...]

        def chunk_body(cc, carry):
            acc_s, acc_q = carry
            for i in range(nb):
                h1 = act_ref[cc * nb + i]
                a1 = jnp.maximum(h1 * s1 + t1, jnp.bfloat16(0.0))
                h2 = jnp.dot(w2c, _shift_stack(a1),
                             preferred_element_type=jnp.float32)
                act_ref[cc * nb + i] = h2.astype(jnp.bfloat16)
                acc_s = acc_s + _fold(h2, sw)
                acc_q = acc_q + _fold(h2 * h2, sw)
            return acc_s, acc_q

        acc_s, acc_q = jax.lax.fori_loop(
            0, nc, chunk_body,
            (jnp.zeros_like(sum_ref[...]), jnp.zeros_like(sq_ref[...])))

        mean2 = jnp.sum(acc_s, axis=1, keepdims=True) / cnt
        var2 = jnp.maximum(
            jnp.sum(acc_q, axis=1, keepdims=True) / cnt - mean2 * mean2, 0.0)
        scale2 = g2_ref[...] * jax.lax.rsqrt(var2 + eps)
        sc_ref[...] = scale2
        sh_ref[...] = b2_ref[...] - mean2 * scale2

    @pl.when(s > nc)
    def _phase2():
        c2 = s - nc - 1
        wpc = wp_ref[...]
        s2 = sc_ref[...]
        t2 = sh_ref[...]
        for i in range(nb):
            proj = jnp.dot(wpc, xb_ref[c2 * nb + i],
                           preferred_element_type=jnp.float32)
            z = act_ref[c2 * nb + i].astype(jnp.float32) * s2 + t2
            o_ref[i] = jnp.maximum(z + proj, 0.0)


def kernel(x, w1, g1, b1, w2, g2, b2, wp, eps=1e-5):
    N, Cin, L = x.shape
    Cout = w1.shape[0]
    K = w1.shape[2]
    assert K == 3, "kernel specialized for k=3 'same' convolutions"

    w1c = jnp.transpose(w1, (0, 2, 1)).reshape(Cout, K * Cin).astype(jnp.bfloat16)
    w2c = jnp.transpose(w2, (0, 2, 1)).reshape(Cout, K * Cout).astype(jnp.bfloat16)
    wpc = wp[:, :, 0].astype(jnp.bfloat16)

    nb = next(n for n in (8, 4, 2, 1) if N % n == 0)
    sw = min(128, L)
    nc = N // nb
    grid = (2 * nc + 1,)
    cparams = pltpu.CompilerParams(
        dimension_semantics=("arbitrary",),
        vmem_limit_bytes=60 * 1024 * 1024,
    )

    def rep3(shape):
        return pl.BlockSpec(tuple(shape), lambda s: (0,) * len(shape))

    x_spec = pl.BlockSpec((nb, Cin, L), lambda s: (jnp.where(s < nc, s, 0), 0, 0))
    o_spec = pl.BlockSpec((nb, Cout, L),
                          lambda s: (jnp.where(s > nc, s - nc - 1, 0), 0, 0))

    out = pl.pallas_call(
        functools.partial(_fused_kernel, nb=nb, nc=nc,
                          cnt=float(N * L), eps=float(eps), sw=sw),
        grid=grid,
        in_specs=[x_spec, rep3(w1c.shape), rep3(w2c.shape), rep3(wpc.shape),
                  rep3((Cout, 1)), rep3((Cout, 1)), rep3((Cout, 1)), rep3((Cout, 1))],
        out_specs=o_spec,
        out_shape=jax.ShapeDtypeStruct((N, Cout, L), jnp.float32),
        scratch_shapes=[
            pltpu.VMEM((N, Cout, L), jnp.bfloat16),
            pltpu.VMEM((N, Cin, L), jnp.bfloat16),
            pltpu.VMEM((Cout, sw), jnp.float32),
            pltpu.VMEM((Cout, sw), jnp.float32),
            pltpu.VMEM((Cout, 1), jnp.float32),
            pltpu.VMEM((Cout, 1), jnp.float32),
        ],
        compiler_params=cparams,
    )(x, w1c, w2c, wpc,
      g1.astype(jnp.float32)[:, None], b1.astype(jnp.float32)[:, None],
      g2.astype(jnp.float32)[:, None], b2.astype(jnp.float32)[:, None])
    return out


# speedup vs baseline: 2.0853x; 1.0070x over previous
import functools

import jax
import jax.numpy as jnp
from jax.experimental import pallas as pl
from jax.experimental.pallas import tpu as pltpu


def _shift_stack(x):
    z = jnp.zeros((x.shape[0], 1), x.dtype)
    xl = jnp.concatenate([z, x[:, :-1]], axis=1)
    xr = jnp.concatenate([x[:, 1:], z], axis=1)
    return jnp.concatenate([xl, x, xr], axis=0)


def _fold(h, w):
    r = h[:, :w]
    for j in range(1, h.shape[1] // w):
        r = r + h[:, j * w:(j + 1) * w]
    return r


def _fused_kernel(x_ref, w1_ref, w2_ref, wp_ref, g1_ref, b1_ref, g2_ref, b2_ref,
                  o_ref, act_ref, xb_ref, sum_ref, sq_ref, sc_ref, sh_ref,
                  *, nb, nb2, nc, cnt, eps, sw):
    s = pl.program_id(0)
    c = s

    @pl.when(s == 0)
    def _zero_bn1():
        sum_ref[...] = jnp.zeros_like(sum_ref)
        sq_ref[...] = jnp.zeros_like(sq_ref)

    @pl.when(s < nc)
    def _phase0():
        w1c = w1_ref[...]
        acc_s = jnp.zeros_like(sum_ref)
        acc_q = jnp.zeros_like(sq_ref)
        for i in range(nb):
            xb = x_ref[i].astype(jnp.bfloat16)
            xb_ref[c * nb + i] = xb
            h1 = jnp.dot(w1c, _shift_stack(xb),
                         preferred_element_type=jnp.float32)
            act_ref[c * nb + i] = h1.astype(jnp.bfloat16)
            acc_s = acc_s + _fold(h1, sw)
            acc_q = acc_q + _fold(h1 * h1, sw)
        sum_ref[...] += acc_s
        sq_ref[...] += acc_q

    @pl.when(s == nc)
    def _phase1():
        mean = jnp.sum(sum_ref[...], axis=1, keepdims=True) / cnt
        var = jnp.maximum(
            jnp.sum(sq_ref[...], axis=1, keepdims=True) / cnt - mean * mean, 0.0)
        scale1 = g1_ref[...] * jax.lax.rsqrt(var + eps)
        s1 = scale1.astype(jnp.bfloat16)
        t1 = (b1_ref[...] - mean * scale1).astype(jnp.bfloat16)

        w2c = w2_ref[...]

        def chunk_body(cc, carry):
            acc_s, acc_q = carry
            for i in range(nb):
                h1 = act_ref[cc * nb + i]
                a1 = jnp.maximum(h1 * s1 + t1, jnp.bfloat16(0.0))
                h2 = jnp.dot(w2c, _shift_stack(a1),
                             preferred_element_type=jnp.float32)
                act_ref[cc * nb + i] = h2.astype(jnp.bfloat16)
                acc_s = acc_s + _fold(h2, sw)
                acc_q = acc_q + _fold(h2 * h2, sw)
            return acc_s, acc_q

        acc_s, acc_q = jax.lax.fori_loop(
            0, nc, chunk_body,
            (jnp.zeros_like(sum_ref[...]), jnp.zeros_like(sq_ref[...])))

        mean2 = jnp.sum(acc_s, axis=1, keepdims=True) / cnt
        var2 = jnp.maximum(
            jnp.sum(acc_q, axis=1, keepdims=True) / cnt - mean2 * mean2, 0.0)
        scale2 = g2_ref[...] * jax.lax.rsqrt(var2 + eps)
        sc_ref[...] = scale2
        sh_ref[...] = b2_ref[...] - mean2 * scale2

    @pl.when(s > nc)
    def _phase2():
        c2 = s - nc - 1
        wpc = wp_ref[...]
        s2 = sc_ref[...]
        t2 = sh_ref[...]
        for i in range(nb2):
            proj = jnp.dot(wpc, xb_ref[c2 * nb2 + i],
                           preferred_element_type=jnp.float32)
            z = act_ref[c2 * nb2 + i].astype(jnp.float32) * s2 + t2
            o_ref[i] = jnp.maximum(z + proj, 0.0)


def kernel(x, w1, g1, b1, w2, g2, b2, wp, eps=1e-5):
    N, Cin, L = x.shape
    Cout = w1.shape[0]
    K = w1.shape[2]
    assert K == 3, "kernel specialized for k=3 'same' convolutions"

    w1c = jnp.transpose(w1, (0, 2, 1)).reshape(Cout, K * Cin).astype(jnp.bfloat16)
    w2c = jnp.transpose(w2, (0, 2, 1)).reshape(Cout, K * Cout).astype(jnp.bfloat16)
    wpc = wp[:, :, 0].astype(jnp.bfloat16)

    nb = next(n for n in (8, 4, 2, 1) if N % n == 0)
    sw = min(128, L)
    nc = N // nb
    nb2 = next(n for n in (16, 8, 4, 2, 1) if N % n == 0)
    nc2 = N // nb2
    grid = (nc + 1 + nc2,)
    cparams = pltpu.CompilerParams(
        dimension_semantics=("arbitrary",),
        vmem_limit_bytes=60 * 1024 * 1024,
    )

    def rep3(shape):
        return pl.BlockSpec(tuple(shape), lambda s: (0,) * len(shape))

    x_spec = pl.BlockSpec((nb, Cin, L), lambda s: (jnp.where(s < nc, s, 0), 0, 0))
    o_spec = pl.BlockSpec((nb2, Cout, L),
                          lambda s: (jnp.where(s > nc, s - nc - 1, 0), 0, 0))

    out = pl.pallas_call(
        functools.partial(_fused_kernel, nb=nb, nc=nc,
                          cnt=float(N * L), eps=float(eps), sw=sw, nb2=nb2),
        grid=grid,
        in_specs=[x_spec, rep3(w1c.shape), rep3(w2c.shape), rep3(wpc.shape),
                  rep3((Cout, 1)), rep3((Cout, 1)), rep3((Cout, 1)), rep3((Cout, 1))],
        out_specs=o_spec,
        out_shape=jax.ShapeDtypeStruct((N, Cout, L), jnp.float32),
        scratch_shapes=[
            pltpu.VMEM((N, Cout, L), jnp.bfloat16),
            pltpu.VMEM((N, Cin, L), jnp.bfloat16),
            pltpu.VMEM((Cout, sw), jnp.float32),
            pltpu.VMEM((Cout, sw), jnp.float32),
            pltpu.VMEM((Cout, 1), jnp.float32),
            pltpu.VMEM((Cout, 1), jnp.float32),
        ],
        compiler_params=cparams,
    )(x, w1c, w2c, wpc,
      g1.astype(jnp.float32)[:, None], b1.astype(jnp.float32)[:, None],
      g2.astype(jnp.float32)[:, None], b2.astype(jnp.float32)[:, None])
    return out


# speedup vs baseline: 2.1807x; 1.0457x over previous
import functools

import jax
import jax.numpy as jnp
from jax.experimental import pallas as pl
from jax.experimental.pallas import tpu as pltpu


def _shift_stack(x):
    z = jnp.zeros((x.shape[0], 1), x.dtype)
    xl = jnp.concatenate([z, x[:, :-1]], axis=1)
    xr = jnp.concatenate([x[:, 1:], z], axis=1)
    return jnp.concatenate([xl, x, xr], axis=0)


def _fold(h, w):
    r = h[:, :w]
    for j in range(1, h.shape[1] // w):
        r = r + h[:, j * w:(j + 1) * w]
    return r


def _fused_kernel(x_ref, w1_ref, w2_ref, wp_ref, g1_ref, b1_ref, g2_ref, b2_ref,
                  o_ref, act_ref, xb_ref, sum_ref, sq_ref, sc_ref, sh_ref,
                  *, nb, nb2, nc, cnt, eps, sw):
    s = pl.program_id(0)
    c = s

    @pl.when(s == 0)
    def _zero_bn1():
        sum_ref[...] = jnp.zeros_like(sum_ref)
        sq_ref[...] = jnp.zeros_like(sq_ref)

    @pl.when(s < nc)
    def _phase0():
        w1c = w1_ref[...]
        acc_s = jnp.zeros_like(sum_ref)
        acc_q = jnp.zeros_like(sq_ref)
        for i in range(nb):
            xb = x_ref[i].astype(jnp.bfloat16)
            xb_ref[c * nb + i] = xb
            h1 = jnp.dot(w1c, _shift_stack(xb),
                         preferred_element_type=jnp.float32)
            act_ref[c * nb + i] = h1.astype(jnp.bfloat16)
            acc_s = acc_s + _fold(h1, sw)
            acc_q = acc_q + _fold(h1 * h1, sw)
        sum_ref[...] += acc_s
        sq_ref[...] += acc_q

    @pl.when(s == nc)
    def _phase1():
        mean = jnp.sum(sum_ref[...], axis=1, keepdims=True) / cnt
        var = jnp.maximum(
            jnp.sum(sq_ref[...], axis=1, keepdims=True) / cnt - mean * mean, 0.0)
        scale1 = g1_ref[...] * jax.lax.rsqrt(var + eps)
        s1 = scale1.astype(jnp.bfloat16)
        t1 = (b1_ref[...] - mean * scale1).astype(jnp.bfloat16)

        w2c = w2_ref[...]

        acc_s = jnp.zeros_like(sum_ref[...])
        acc_q = jnp.zeros_like(sq_ref[...])
        for j in range(nc * nb):
            h1 = act_ref[j]
            a1 = jnp.maximum(h1 * s1 + t1, jnp.bfloat16(0.0))
            h2 = jnp.dot(w2c, _shift_stack(a1),
                         preferred_element_type=jnp.float32)
            act_ref[j] = h2.astype(jnp.bfloat16)
            acc_s = acc_s + _fold(h2, sw)
            acc_q = acc_q + _fold(h2 * h2, sw)

        mean2 = jnp.sum(acc_s, axis=1, keepdims=True) / cnt
        var2 = jnp.maximum(
            jnp.sum(acc_q, axis=1, keepdims=True) / cnt - mean2 * mean2, 0.0)
        scale2 = g2_ref[...] * jax.lax.rsqrt(var2 + eps)
        sc_ref[...] = scale2
        sh_ref[...] = b2_ref[...] - mean2 * scale2

    @pl.when(s > nc)
    def _phase2():
        c2 = s - nc - 1
        wpc = wp_ref[...]
        s2 = sc_ref[...]
        t2 = sh_ref[...]
        for i in range(nb2):
            proj = jnp.dot(wpc, xb_ref[c2 * nb2 + i],
                           preferred_element_type=jnp.float32)
            z = act_ref[c2 * nb2 + i].astype(jnp.float32) * s2 + t2
            o_ref[i] = jnp.maximum(z + proj, 0.0)


def kernel(x, w1, g1, b1, w2, g2, b2, wp, eps=1e-5):
    N, Cin, L = x.shape
    Cout = w1.shape[0]
    K = w1.shape[2]
    assert K == 3, "kernel specialized for k=3 'same' convolutions"

    w1c = jnp.transpose(w1, (0, 2, 1)).reshape(Cout, K * Cin).astype(jnp.bfloat16)
    w2c = jnp.transpose(w2, (0, 2, 1)).reshape(Cout, K * Cout).astype(jnp.bfloat16)
    wpc = wp[:, :, 0].astype(jnp.bfloat16)

    nb = next(n for n in (8, 4, 2, 1) if N % n == 0)
    sw = min(128, L)
    nc = N // nb
    nb2 = next(n for n in (16, 8, 4, 2, 1) if N % n == 0)
    nc2 = N // nb2
    grid = (nc + 1 + nc2,)
    cparams = pltpu.CompilerParams(
        dimension_semantics=("arbitrary",),
        vmem_limit_bytes=60 * 1024 * 1024,
    )

    def rep3(shape):
        return pl.BlockSpec(tuple(shape), lambda s: (0,) * len(shape))

    x_spec = pl.BlockSpec((nb, Cin, L), lambda s: (jnp.where(s < nc, s, 0), 0, 0))
    o_spec = pl.BlockSpec((nb2, Cout, L),
                          lambda s: (jnp.where(s > nc, s - nc - 1, 0), 0, 0))

    out = pl.pallas_call(
        functools.partial(_fused_kernel, nb=nb, nc=nc,
                          cnt=float(N * L), eps=float(eps), sw=sw, nb2=nb2),
        grid=grid,
        in_specs=[x_spec, rep3(w1c.shape), rep3(w2c.shape), rep3(wpc.shape),
                  rep3((Cout, 1)), rep3((Cout, 1)), rep3((Cout, 1)), rep3((Cout, 1))],
        out_specs=o_spec,
        out_shape=jax.ShapeDtypeStruct((N, Cout, L), jnp.float32),
        scratch_shapes=[
            pltpu.VMEM((N, Cout, L), jnp.bfloat16),
            pltpu.VMEM((N, Cin, L), jnp.bfloat16),
            pltpu.VMEM((Cout, sw), jnp.float32),
            pltpu.VMEM((Cout, sw), jnp.float32),
            pltpu.VMEM((Cout, 1), jnp.float32),
            pltpu.VMEM((Cout, 1), jnp.float32),
        ],
        compiler_params=cparams,
    )(x, w1c, w2c, wpc,
      g1.astype(jnp.float32)[:, None], b1.astype(jnp.float32)[:, None],
      g2.astype(jnp.float32)[:, None], b2.astype(jnp.float32)[:, None])
    return out
